# Initial kernel scaffold; baseline (speedup 1.0000x reference)
import jax
import jax.numpy as jnp
from jax.experimental import pallas as pl


def kernel(x, edge_index, batch, W_enc, b_enc, W_dec, b_dec):
    raise NotImplementedError("write your pallas kernel here")


# speedup vs baseline: 21.5877x; 21.5877x over previous
import functools

import jax
import jax.numpy as jnp
from jax import lax
from jax.experimental import pallas as pl
from jax.experimental.pallas import tpu as pltpu
from jax.experimental.pallas import tpu_sc as plsc

NC = 2
NS = 16
NW = NC * NS
CH = 128

_SC_PARAMS = pltpu.CompilerParams(use_tc_tiling_on_sc=False)


def _sc_degree(dst_pad, zeros8, ones8, R, iters, e_per_w):
    rpt = R // NS
    mesh = plsc.VectorSubcoreMesh(core_axis_name="c", subcore_axis_name="s")

    @functools.partial(
        pl.kernel,
        out_type=jax.ShapeDtypeStruct((NC, R, 8), jnp.float32),
        mesh=mesh,
        scratch_types=[
            pltpu.VMEM((CH,), jnp.int32),
            pltpu.VMEM((CH, 8), jnp.float32),
            pltpu.VMEM((rpt, 8), jnp.float32),
            pltpu.VMEM_SHARED((R, 8), jnp.float32),
        ],
        compiler_params=_SC_PARAMS,
    )
    def k(dst_hbm, zeros_hbm, ones_hbm, out_hbm, idx_v, ones_v, chunk_v, hist_sh):
        cid = lax.axis_index("c")
        sid = lax.axis_index("s")
        wid = sid * NC + cid
        pltpu.sync_copy(zeros_hbm.at[pl.ds(sid * rpt, rpt)], chunk_v)
        pltpu.sync_copy(chunk_v, hist_sh.at[pl.ds(sid * rpt, rpt)])
        pltpu.sync_copy(ones_hbm, ones_v)
        plsc.subcore_barrier()

        def body(i, carry):
            base = wid * e_per_w + i * CH
            pltpu.sync_copy(dst_hbm.at[pl.ds(base, CH)], idx_v)
            pltpu.sync_copy(ones_v, hist_sh.at[idx_v], add=True)
            return carry

        lax.fori_loop(0, iters, body, 0)
        plsc.subcore_barrier()
        pltpu.sync_copy(hist_sh.at[pl.ds(sid * rpt, rpt)], chunk_v)
        pltpu.sync_copy(chunk_v, out_hbm.at[cid, pl.ds(sid * rpt, rpt)])

    return k(dst_pad, zeros8, ones8)


def _sc_edge_pass(src_pad, dst_pad, table, zeros, R, W, shift, iters, e_per_w):
    rpt = R // NS
    mesh = plsc.VectorSubcoreMesh(core_axis_name="c", subcore_axis_name="s")

    @functools.partial(
        pl.kernel,
        out_type=jax.ShapeDtypeStruct((NC, R, W), jnp.float32),
        mesh=mesh,
        scratch_types=[
            pltpu.VMEM((CH,), jnp.int32),
            pltpu.VMEM((CH,), jnp.int32),
            pltpu.VMEM((CH, W), jnp.float32),
            pltpu.VMEM((rpt, W), jnp.float32),
            pltpu.VMEM_SHARED((R, W), jnp.float32),
            pltpu.SemaphoreType.DMA,
        ],
        compiler_params=_SC_PARAMS,
    )
    def k(src_hbm, dst_hbm, table_hbm, zeros_hbm, out_hbm,
          idxs_v, idxd_v, rows_v, chunk_v, acc_sh, sem):
        cid = lax.axis_index("c")
        sid = lax.axis_index("s")
        wid = sid * NC + cid
        pltpu.sync_copy(zeros_hbm.at[pl.ds(sid * rpt, rpt)], chunk_v)
        pltpu.sync_copy(chunk_v, acc_sh.at[pl.ds(sid * rpt, rpt)])
        plsc.subcore_barrier()

        def body(i, carry):
            base = wid * e_per_w + i * CH
            pltpu.sync_copy(src_hbm.at[pl.ds(base, CH)], idxs_v)
            pltpu.sync_copy(dst_hbm.at[pl.ds(base, CH)], idxd_v)
            if shift:
                for j in range(CH // 16):
                    sl = pl.ds(j * 16, 16)
                    idxs_v[sl] = idxs_v[sl] >> 1
                    idxd_v[sl] = idxd_v[sl] >> 1
            pltpu.async_copy(table_hbm.at[idxs_v], rows_v, sem).wait()
            pltpu.sync_copy(rows_v, acc_sh.at[idxd_v], add=True)
            return carry

        lax.fori_loop(0, iters, body, 0)
        plsc.subcore_barrier()
        pltpu.sync_copy(acc_sh.at[pl.ds(sid * rpt, rpt)], chunk_v)
        pltpu.sync_copy(chunk_v, out_hbm.at[cid, pl.ds(sid * rpt, rpt)])

    return k(src_pad, dst_pad, table, zeros)


def _tc_prep_enc(x_pad, W, p0, p1, B=640):
    R, D = x_pad.shape
    H = W.shape[1]

    def body(x_ref, w_ref, p0_ref, p1_ref, o_ref):
        dinv = lax.rsqrt(p0_ref[...] + p1_ref[...] + 1.0)
        o_ref[...] = jnp.dot(x_ref[...], w_ref[...],
                             preferred_element_type=jnp.float32) * dinv

    return pl.pallas_call(
        body,
        grid=(R // B,),
        in_specs=[
            pl.BlockSpec((B, D), lambda i: (i, 0)),
            pl.BlockSpec((D, H), lambda i: (0, 0)),
            pl.BlockSpec((B, 1), lambda i: (i, 0)),
            pl.BlockSpec((B, 1), lambda i: (i, 0)),
        ],
        out_specs=pl.BlockSpec((B, H), lambda i: (i, 0)),
        out_shape=jax.ShapeDtypeStruct((R, H), jnp.float32),
    )(x_pad, W, p0, p1)


def _tc_post_enc(a0, a1, hs, p0, p1, b, B=640):
    R, H = hs.shape

    def body(a0_ref, a1_ref, hs_ref, p0_ref, p1_ref, b_ref, o_ref):
        dinv = lax.rsqrt(p0_ref[...] + p1_ref[...] + 1.0)
        s = (a0_ref[...] + a1_ref[...] + hs_ref[...]) * dinv + b_ref[...]
        o_ref[...] = jnp.maximum(s, 0.0)

    return pl.pallas_call(
        body,
        grid=(R // B,),
        in_specs=[
            pl.BlockSpec((B, H), lambda i: (i, 0)),
            pl.BlockSpec((B, H), lambda i: (i, 0)),
            pl.BlockSpec((B, H), lambda i: (i, 0)),
            pl.BlockSpec((B, 1), lambda i: (i, 0)),
            pl.BlockSpec((B, 1), lambda i: (i, 0)),
            pl.BlockSpec((1, H), lambda i: (0, 0)),
        ],
        out_specs=pl.BlockSpec((B, H), lambda i: (i, 0)),
        out_shape=jax.ShapeDtypeStruct((R, H), jnp.float32),
    )(a0, a1, hs, p0, p1, b)


def _tc_prep_dec(h2, W, q0, q1, B=640):
    Rc, H2 = h2.shape
    H = H2 // 2
    D = W.shape[1]

    def body(h_ref, w_ref, q0_ref, q1_ref, o_ref):
        degc = (q0_ref[:, 0:1] + q0_ref[:, 1:2]
                + q1_ref[:, 0:1] + q1_ref[:, 1:2] + 1.0)
        xc = 0.5 * (h_ref[:, :H] + h_ref[:, H:])
        o_ref[...] = jnp.dot(xc, w_ref[...],
                             preferred_element_type=jnp.float32) * lax.rsqrt(degc)

    return pl.pallas_call(
        body,
        grid=(Rc // B,),
        in_specs=[
            pl.BlockSpec((B, H2), lambda i: (i, 0)),
            pl.BlockSpec((H, D), lambda i: (0, 0)),
            pl.BlockSpec((B, 2), lambda i: (i, 0)),
            pl.BlockSpec((B, 2), lambda i: (i, 0)),
        ],
        out_specs=pl.BlockSpec((B, D), lambda i: (i, 0)),
        out_shape=jax.ShapeDtypeStruct((Rc, D), jnp.float32),
    )(h2, W, q0, q1)


def _tc_final(a0, a1, hds, q0, q1, b, B=640):
    Rc, D = hds.shape

    def body(a0_ref, a1_ref, hds_ref, q0_ref, q1_ref, b_ref, o_ref):
        degc = (q0_ref[:, 0:1] + q0_ref[:, 1:2]
                + q1_ref[:, 0:1] + q1_ref[:, 1:2] + 1.0)
        xd = ((a0_ref[...] + a1_ref[...] + hds_ref[...]) * lax.rsqrt(degc)
              + b_ref[...])
        o_ref[:, :D] = xd
        o_ref[:, D:] = xd

    return pl.pallas_call(
        body,
        grid=(Rc // B,),
        in_specs=[
            pl.BlockSpec((B, D), lambda i: (i, 0)),
            pl.BlockSpec((B, D), lambda i: (i, 0)),
            pl.BlockSpec((B, D), lambda i: (i, 0)),
            pl.BlockSpec((B, 2), lambda i: (i, 0)),
            pl.BlockSpec((B, 2), lambda i: (i, 0)),
            pl.BlockSpec((1, D), lambda i: (0, 0)),
        ],
        out_specs=pl.BlockSpec((B, 2 * D), lambda i: (i, 0)),
        out_shape=jax.ShapeDtypeStruct((Rc, 2 * D), jnp.float32),
    )(a0, a1, hds, q0, q1, b)


def kernel(x, edge_index, batch, W_enc, b_enc, W_dec, b_dec):
    N, D = x.shape
    H = W_enc.shape[1]
    E = edge_index.shape[1]
    Nc = N // 2

    Rc = ((Nc + 1 + 255) // 256) * 256
    R = 2 * Rc
    iters = -(-E // (NW * CH))
    e_per_w = iters * CH
    pad_e = NW * e_per_w - E

    src = jnp.concatenate(
        [edge_index[0], jnp.full((pad_e,), N, jnp.int32)])
    dst = jnp.concatenate(
        [edge_index[1], jnp.full((pad_e,), N, jnp.int32)])

    zeros8 = jnp.zeros((R, 8), jnp.float32)
    ones8 = jnp.zeros((CH, 8), jnp.float32).at[:, 0].set(1.0)
    zf = jnp.zeros((R, H), jnp.float32)
    zc = jnp.zeros((Rc, D), jnp.float32)
    x_pad = jnp.concatenate([x, jnp.zeros((R - N, D), x.dtype)])

    degp = _sc_degree(dst, zeros8, ones8, R, iters, e_per_w)
    p0 = degp[0, :, :1]
    p1 = degp[1, :, :1]

    hs = _tc_prep_enc(x_pad, W_enc, p0, p1)
    accf = _sc_edge_pass(src, dst, hs, zf, R, H, False, iters, e_per_w)
    h_enc = _tc_post_enc(accf[0], accf[1], hs, p0, p1, b_enc.reshape(1, H))

    q0 = p0.reshape(Rc, 2)
    q1 = p1.reshape(Rc, 2)
    hds = _tc_prep_dec(h_enc.reshape(Rc, 2 * H), W_dec, q0, q1)
    accc = _sc_edge_pass(src, dst, hds, zc, Rc, D, True, iters, e_per_w)
    outd = _tc_final(accc[0], accc[1], hds, q0, q1, b_dec.reshape(1, D))

    return outd[:Nc].reshape(N, D)
